# baseline (speedup 1.0000x reference)
import functools

import jax
import jax.numpy as jnp
from jax import lax
from jax.experimental import pallas as pl
from jax.experimental.pallas import tpu as pltpu
from jax.experimental.pallas import tpu_sc as plsc

N = 10000
E = 320000
B = 16
D = 128
NCLS = 10
SEQ_D = 1280

NCORES = 2
NSUB = 16
NTILES = NCORES * NSUB
EB = 128
NB0 = 98
NB1 = 64
NBMAX = max(NB0, NB1)
EROWS = NSUB * (NB0 + NB1)
EPAD = EROWS * EB
NPAD = 10240
NP = 10112
RPT = NP // NSUB
DCH = NPAD // NSUB

@functools.cache
def _mesh():
    return plsc.VectorSubcoreMesh(core_axis_name="c", subcore_axis_name="s",
                                  num_cores=NCORES, num_subcores=NSUB)


def _splat(v, i):
    idx = jnp.full((16, 1), i, jnp.int32)
    dnums = lax.GatherDimensionNumbers(
        offset_dims=(), collapsed_slice_dims=(0,), start_index_map=(0,))
    return lax.gather(v, idx, dnums, (1,),
                      mode=lax.GatherScatterMode.PROMISE_IN_BOUNDS)


def _deg_body(dst_hbm, ew_hbm, out_hbm, acc_sh, dstb, ewb, zb):
    c = lax.axis_index("c")
    s = lax.axis_index("s")
    wid = c * NSUB + s
    nb = lax.select(c == 0, NB0, NB1)
    pltpu.sync_copy(dst_hbm.at[wid], dstb)
    pltpu.sync_copy(ew_hbm.at[wid], ewb)
    zero = jnp.zeros((16,), jnp.float32)

    def zrow(k, carry):
        zb[pl.ds(k * 16, 16)] = zero
        return carry

    lax.fori_loop(0, DCH // 16, zrow, 0)
    pltpu.sync_copy(zb, acc_sh.at[pl.ds(s * DCH, DCH)])
    plsc.subcore_barrier()

    def batch(j, carry):
        pltpu.sync_copy(ewb.at[j], acc_sh.at[dstb.at[j]], add=True)
        return carry

    lax.fori_loop(0, nb, batch, 0)
    plsc.subcore_barrier()
    pltpu.sync_copy(acc_sh.at[pl.ds(s * DCH, DCH)],
                    out_hbm.at[c, pl.ds(s * DCH, DCH)])


@functools.cache
def _deg_kernel():
    return pl.kernel(
        _deg_body,
        out_type=jax.ShapeDtypeStruct((NCORES, NPAD), jnp.float32),
        mesh=_mesh(),
        scratch_types=[
            pltpu.VMEM_SHARED((NPAD,), jnp.float32),
            pltpu.VMEM((NBMAX, EB), jnp.int32),
            pltpu.VMEM((NBMAX, EB), jnp.float32),
            pltpu.VMEM((DCH,), jnp.float32),
        ],
    )


def _scatter_body(hp_hbm, src_hbm, ew_hbm, dst_hbm, out_hbm,
                  acc_sh, dstb, ring, ringw, rbuf0, rbuf1, gsem0, gsem1, isem,
                  ssem0, ssem1):
    c = lax.axis_index("c")
    s = lax.axis_index("s")
    wid = c * NSUB + s
    nb = lax.select(c == 0, NB0, NB1)
    pltpu.sync_copy(dst_hbm.at[wid], dstb)

    zero = jnp.zeros((16,), jnp.float32)

    def zrow(r, carry):
        for f in range(8):
            rbuf0[r, pl.ds(f * 16, 16)] = zero
        return carry

    lax.fori_loop(0, EB, zrow, 0)
    for k in range(RPT // EB):
        pltpu.sync_copy(rbuf0, acc_sh.at[pl.ds(s * RPT + k * EB, EB)])
    rem = RPT - (RPT // EB) * EB
    if rem:
        pltpu.sync_copy(rbuf0.at[pl.ds(0, rem)],
                        acc_sh.at[pl.ds(s * RPT + (RPT // EB) * EB, rem)])
    plsc.subcore_barrier()

    pltpu.sync_copy(src_hbm.at[wid, 0], ring.at[0])
    pltpu.sync_copy(ew_hbm.at[wid, 0], ringw.at[0])
    pltpu.async_copy(hp_hbm.at[ring.at[0]], rbuf0, gsem0)
    pltpu.async_copy(src_hbm.at[wid, 1], ring.at[1], isem)
    pltpu.async_copy(ew_hbm.at[wid, 1], ringw.at[1], isem)

    def scale(jp, rbuf):
        @plsc.parallel_loop(0, EB // 16, 1, unroll=2)
        def grp(g):
            nv = ringw[jp, pl.ds(g * 16, 16)]
            for i in range(16):
                sp = _splat(nv, i)
                e = g * 16 + i
                for f in range(8):
                    rbuf[e, pl.ds(f * 16, 16)] = rbuf[e, pl.ds(f * 16, 16)] * sp

    def step(j, rbuf, obuf, osem, ssem_self, ssem_other):
        p = j % 2

        @pl.when(j + 1 < nb)
        def _():
            pltpu.make_async_copy(src_hbm.at[wid, j + 1], ring.at[1 - p],
                                  isem).wait()
            pltpu.make_async_copy(ew_hbm.at[wid, j + 1], ringw.at[1 - p],
                                  isem).wait()

            @pl.when(j >= 1)
            def _():
                pltpu.make_async_copy(obuf, acc_sh.at[dstb.at[j - 1]],
                                      ssem_other).wait()

            pltpu.async_copy(hp_hbm.at[ring.at[1 - p]], obuf, osem)

        scale(p, rbuf)

        @pl.when(j + 2 < nb)
        def _():
            pltpu.async_copy(src_hbm.at[wid, j + 2], ring.at[p], isem)
            pltpu.async_copy(ew_hbm.at[wid, j + 2], ringw.at[p], isem)

        pltpu.async_copy(rbuf, acc_sh.at[dstb.at[j]], ssem_self, add=True)

    def pair(jp, carry):
        j0 = jp * 2
        pltpu.make_async_copy(hp_hbm.at[ring.at[0]], rbuf0, gsem0).wait()
        step(j0, rbuf0, rbuf1, gsem1, ssem0, ssem1)
        pltpu.make_async_copy(hp_hbm.at[ring.at[1]], rbuf1, gsem1).wait()
        step(j0 + 1, rbuf1, rbuf0, gsem0, ssem1, ssem0)
        return carry

    lax.fori_loop(0, nb // 2, pair, 0)

    pltpu.make_async_copy(rbuf0, acc_sh.at[dstb.at[nb - 2]], ssem0).wait()
    pltpu.make_async_copy(rbuf1, acc_sh.at[dstb.at[nb - 1]], ssem1).wait()

    plsc.subcore_barrier()
    pltpu.sync_copy(acc_sh.at[pl.ds(s * RPT, RPT)],
                    out_hbm.at[c, pl.ds(s * RPT, RPT)])


@functools.cache
def _scatter_kernel():
    return pl.kernel(
        _scatter_body,
        out_type=jax.ShapeDtypeStruct((NCORES, NP, D), jnp.float32),
        mesh=_mesh(),
        scratch_types=[
            pltpu.VMEM_SHARED((NP, D), jnp.float32),
            pltpu.VMEM((NBMAX, EB), jnp.int32),
            pltpu.VMEM((2, EB), jnp.int32),
            pltpu.VMEM((2, EB), jnp.float32),
            pltpu.VMEM((EB, D), jnp.float32),
            pltpu.VMEM((EB, D), jnp.float32),
            pltpu.SemaphoreType.DMA,
            pltpu.SemaphoreType.DMA,
            pltpu.SemaphoreType.DMA,
            pltpu.SemaphoreType.DMA,
            pltpu.SemaphoreType.DMA,
        ],
    )


_BLK = 1000
_NBLK = N // _BLK


def _dot(a, b):
    return jnp.dot(a, b, preferred_element_type=jnp.float32,
                   precision=lax.Precision.HIGHEST)


def _prep_body(d0_ref, d1_ref, x_ref, w_ref, disc_ref, o_ref):
    disc = lax.rsqrt(d0_ref[...] + d1_ref[...])
    disc_ref[...] = disc
    o_ref[...] = disc * _dot(x_ref[...], w_ref[...])


def _tc_prep(d0, d1, x, w):
    return pl.pallas_call(
        _prep_body,
        grid=(_NBLK,),
        in_specs=[
            pl.BlockSpec((_BLK, 1), lambda i: (i, 0)),
            pl.BlockSpec((_BLK, 1), lambda i: (i, 0)),
            pl.BlockSpec((_BLK, D), lambda i: (i, 0)),
            pl.BlockSpec((D, D), lambda i: (0, 0)),
        ],
        out_specs=[
            pl.BlockSpec((_BLK, 1), lambda i: (i, 0)),
            pl.BlockSpec((_BLK, D), lambda i: (i, 0)),
        ],
        out_shape=[
            jax.ShapeDtypeStruct((N, 1), jnp.float32),
            jax.ShapeDtypeStruct((N, D), jnp.float32),
        ],
    )(d0, d1, x, w)


def _bn_from(sacc, t, g_ref, be_ref):
    mu = sacc[0:1, :] * (1.0 / N)
    var = sacc[1:2, :] * (1.0 / N) - mu * mu
    return (t - mu) * lax.rsqrt(var + 1e-5) * g_ref[...] + be_ref[...]


def _layer_body(a0_ref, a1_ref, disc_ref, b_ref, g_ref, be_ref, w_ref,
                o_ref, tbuf, sacc, *, relu):
    i = pl.program_id(0)

    @pl.when(i == 0)
    def _():
        sacc[...] = jnp.zeros_like(sacc)

    @pl.when(i < _NBLK)
    def _():
        t = disc_ref[...] * (a0_ref[...] + a1_ref[...]) + b_ref[...]
        if relu:
            t = jnp.maximum(t, 0.0)
        k = i * _BLK
        tbuf[pl.ds(k, _BLK), :] = t
        sacc[0:1, :] += jnp.sum(t, axis=0, keepdims=True)
        sacc[1:2, :] += jnp.sum(t * t, axis=0, keepdims=True)

    @pl.when(i >= _NBLK)
    def _():
        k = (i - _NBLK) * _BLK
        xn = _bn_from(sacc, tbuf[pl.ds(k, _BLK), :], g_ref, be_ref)
        o_ref[...] = disc_ref[...] * _dot(xn, w_ref[...])


def _tc_layer(a0, a1, disc, bias, g, be, w, relu):
    return pl.pallas_call(
        functools.partial(_layer_body, relu=relu),
        grid=(2 * _NBLK,),
        in_specs=[
            pl.BlockSpec((_BLK, D), lambda i: (i % _NBLK, 0)),
            pl.BlockSpec((_BLK, D), lambda i: (i % _NBLK, 0)),
            pl.BlockSpec((_BLK, 1), lambda i: (i % _NBLK, 0)),
            pl.BlockSpec((1, D), lambda i: (0, 0)),
            pl.BlockSpec((1, D), lambda i: (0, 0)),
            pl.BlockSpec((1, D), lambda i: (0, 0)),
            pl.BlockSpec((D, D), lambda i: (0, 0)),
        ],
        out_specs=pl.BlockSpec((_BLK, D), lambda i: (i % _NBLK, 0)),
        out_shape=jax.ShapeDtypeStruct((N, D), jnp.float32),
        scratch_shapes=[
            pltpu.VMEM((N, D), jnp.float32),
            pltpu.VMEM((2, D), jnp.float32),
        ],
    )(a0, a1, disc, bias, g, be, w)


def _tail_body(a0_ref, a1_ref, disc_ref, b_ref, g_ref, be_ref, bat_ref,
               seq_ref, fw_ref, fb_ref, lw_ref, lb_ref, o_ref,
               tbuf, sacc, pool_s, cnt_s):
    i = pl.program_id(0)

    @pl.when(i == 0)
    def _():
        sacc[...] = jnp.zeros_like(sacc)
        pool_s[...] = jnp.zeros_like(pool_s)
        for cc in range(B):
            cnt_s[0, cc] = 0.0

    @pl.when(i < _NBLK)
    def _():
        t = disc_ref[...] * (a0_ref[...] + a1_ref[...]) + b_ref[...]
        k = i * _BLK
        tbuf[pl.ds(k, _BLK), :] = t
        sacc[0:1, :] += jnp.sum(t, axis=0, keepdims=True)
        sacc[1:2, :] += jnp.sum(t * t, axis=0, keepdims=True)

    @pl.when(i >= _NBLK)
    def _():
        k = (i - _NBLK) * _BLK
        xn = _bn_from(sacc, tbuf[pl.ds(k, _BLK), :], g_ref, be_ref)
        bat = bat_ref[0]
        for cc in range(B):
            m = (bat == cc).astype(jnp.float32)
            pool_s[cc:cc + 1, :] += _dot(m, xn)
            cnt_s[0, cc] += jnp.sum(m)

    @pl.when(i == 2 * _NBLK - 1)
    def _():
        for cc in range(B):
            inv = 1.0 / jnp.maximum(cnt_s[0, cc], 1.0)
            pool_s[cc:cc + 1, :] *= inv
        z = pool_s[...] + _dot(seq_ref[...], fw_ref[...]) + fb_ref[...]
        o_ref[...] = jax.nn.sigmoid(_dot(z, lw_ref[...]) + lb_ref[...])


def _tc_tail(a0, a1, disc, bias, g, be, bat3d, seq, fw, fb, lw, lb):
    return pl.pallas_call(
        _tail_body,
        grid=(2 * _NBLK,),
        in_specs=[
            pl.BlockSpec((_BLK, D), lambda i: (i % _NBLK, 0)),
            pl.BlockSpec((_BLK, D), lambda i: (i % _NBLK, 0)),
            pl.BlockSpec((_BLK, 1), lambda i: (i % _NBLK, 0)),
            pl.BlockSpec((1, D), lambda i: (0, 0)),
            pl.BlockSpec((1, D), lambda i: (0, 0)),
            pl.BlockSpec((1, D), lambda i: (0, 0)),
            pl.BlockSpec((1, 1, _BLK), lambda i: (i % _NBLK, 0, 0)),
            pl.BlockSpec((B, SEQ_D), lambda i: (0, 0)),
            pl.BlockSpec((SEQ_D, D), lambda i: (0, 0)),
            pl.BlockSpec((1, D), lambda i: (0, 0)),
            pl.BlockSpec((D, D), lambda i: (0, 0)),
            pl.BlockSpec((1, D), lambda i: (0, 0)),
        ],
        out_specs=pl.BlockSpec((B, D), lambda i: (0, 0)),
        out_shape=jax.ShapeDtypeStruct((B, D), jnp.float32),
        scratch_shapes=[
            pltpu.VMEM((N, D), jnp.float32),
            pltpu.VMEM((2, D), jnp.float32),
            pltpu.VMEM((B, D), jnp.float32),
            pltpu.SMEM((1, B), jnp.float32),
        ],
    )(a0, a1, disc, bias, g, be, bat3d, seq, fw, fb, lw, lb)


def kernel(embedding_features_per_residue, edge_index, edge_attr, batch,
           embedding_features_per_sequence, W1, b1, W2, b2, W3, b3,
           g1, be1, g2, be2, g3, be3, fc1_W, fc1_b, lin_W, lin_b):
    x = embedding_features_per_residue
    src = edge_index[0]
    dst = edge_index[1]
    ew = edge_attr[:, 0]

    loop = jnp.arange(N, dtype=jnp.int32)
    padi = jnp.zeros((EPAD - E - N,), jnp.int32)
    srcF = jnp.concatenate([src, loop, padi])
    dstF = jnp.concatenate([dst, loop, padi])
    ewF = jnp.concatenate([ew, jnp.ones((N,), jnp.float32),
                           jnp.zeros((EPAD - E - N,), jnp.float32)])

    def _tileize(flat):
        arr = jnp.zeros((NTILES, NBMAX, EB), flat.dtype)
        off = 0
        for t in range(NTILES):
            nb = NB0 if t < NSUB else NB1
            arr = arr.at[t, :nb].set(flat[off:off + nb * EB].reshape(nb, EB))
            off += nb * EB
        return arr

    srcA = _tileize(srcF)
    dstA = _tileize(dstF)
    ewA = _tileize(ewF)

    dega = _deg_kernel()(dstA, ewA)
    d0 = dega[0, :N].reshape(N, 1)
    d1 = dega[1, :N].reshape(N, 1)

    b1r, b2r, b3r = (v.reshape(1, D) for v in (b1, b2, b3))
    g1r, g2r, g3r = (v.reshape(1, D) for v in (g1, g2, g3))
    be1r, be2r, be3r = (v.reshape(1, D) for v in (be1, be2, be3))

    disc, h = _tc_prep(d0, d1, x, W1)

    acc = _scatter_kernel()(h, srcA, ewA, dstA)
    h = _tc_layer(acc[0, :N], acc[1, :N], disc, b1r, g1r, be1r, W2, relu=True)

    acc = _scatter_kernel()(h, srcA, ewA, dstA)
    h = _tc_layer(acc[0, :N], acc[1, :N], disc, b2r, g2r, be2r, W3, relu=True)

    acc = _scatter_kernel()(h, srcA, ewA, dstA)

    bat3d = batch.reshape(_NBLK, 1, _BLK)
    lwp = jnp.zeros((D, D), jnp.float32).at[:, :NCLS].set(lin_W)
    lbp = jnp.zeros((1, D), jnp.float32).at[0, :NCLS].set(lin_b)
    out = _tc_tail(acc[0, :N], acc[1, :N], disc, b3r, g3r, be3r, bat3d,
                   embedding_features_per_sequence, fc1_W,
                   fc1_b.reshape(1, D), lwp, lbp)
    return out[:, :NCLS]


# speedup vs baseline: 1.0159x; 1.0159x over previous
import functools

import jax
import jax.numpy as jnp
from jax import lax
from jax.experimental import pallas as pl
from jax.experimental.pallas import tpu as pltpu
from jax.experimental.pallas import tpu_sc as plsc

N = 10000
E = 320000
B = 16
D = 128
NCLS = 10
SEQ_D = 1280

NCORES = 2
NSUB = 16
NTILES = NCORES * NSUB
EB = 128
NB0 = 106
NB1 = 56
NBMAX = max(NB0, NB1)
EROWS = NSUB * (NB0 + NB1)
EPAD = EROWS * EB
NPAD = 10240
NP = 10112
RPT = NP // NSUB
DCH = NPAD // NSUB

@functools.cache
def _mesh():
    return plsc.VectorSubcoreMesh(core_axis_name="c", subcore_axis_name="s",
                                  num_cores=NCORES, num_subcores=NSUB)


def _splat(v, i):
    idx = jnp.full((16, 1), i, jnp.int32)
    dnums = lax.GatherDimensionNumbers(
        offset_dims=(), collapsed_slice_dims=(0,), start_index_map=(0,))
    return lax.gather(v, idx, dnums, (1,),
                      mode=lax.GatherScatterMode.PROMISE_IN_BOUNDS)


def _deg_body(dst_hbm, ew_hbm, out_hbm, acc_sh, dstb, ewb, zb):
    c = lax.axis_index("c")
    s = lax.axis_index("s")
    wid = c * NSUB + s
    nb = lax.select(c == 0, NB0, NB1)
    pltpu.sync_copy(dst_hbm.at[wid], dstb)
    pltpu.sync_copy(ew_hbm.at[wid], ewb)
    zero = jnp.zeros((16,), jnp.float32)

    def zrow(k, carry):
        zb[pl.ds(k * 16, 16)] = zero
        return carry

    lax.fori_loop(0, DCH // 16, zrow, 0)
    pltpu.sync_copy(zb, acc_sh.at[pl.ds(s * DCH, DCH)])
    plsc.subcore_barrier()

    def batch(j, carry):
        pltpu.sync_copy(ewb.at[j], acc_sh.at[dstb.at[j]], add=True)
        return carry

    lax.fori_loop(0, nb, batch, 0)
    plsc.subcore_barrier()
    pltpu.sync_copy(acc_sh.at[pl.ds(s * DCH, DCH)],
                    out_hbm.at[c, pl.ds(s * DCH, DCH)])


@functools.cache
def _deg_kernel():
    return pl.kernel(
        _deg_body,
        out_type=jax.ShapeDtypeStruct((NCORES, NPAD), jnp.float32),
        mesh=_mesh(),
        scratch_types=[
            pltpu.VMEM_SHARED((NPAD,), jnp.float32),
            pltpu.VMEM((NBMAX, EB), jnp.int32),
            pltpu.VMEM((NBMAX, EB), jnp.float32),
            pltpu.VMEM((DCH,), jnp.float32),
        ],
    )


def _scatter_body(hp_hbm, src_hbm, ew_hbm, dst_hbm, out_hbm,
                  acc_sh, dstb, ring, ringw, rbuf0, rbuf1, gsem0, gsem1, isem,
                  ssem0, ssem1):
    c = lax.axis_index("c")
    s = lax.axis_index("s")
    wid = c * NSUB + s
    nb = lax.select(c == 0, NB0, NB1)
    pltpu.sync_copy(dst_hbm.at[wid], dstb)

    zero = jnp.zeros((16,), jnp.float32)

    def zrow(r, carry):
        for f in range(8):
            rbuf0[r, pl.ds(f * 16, 16)] = zero
        return carry

    lax.fori_loop(0, EB, zrow, 0)
    for k in range(RPT // EB):
        pltpu.sync_copy(rbuf0, acc_sh.at[pl.ds(s * RPT + k * EB, EB)])
    rem = RPT - (RPT // EB) * EB
    if rem:
        pltpu.sync_copy(rbuf0.at[pl.ds(0, rem)],
                        acc_sh.at[pl.ds(s * RPT + (RPT // EB) * EB, rem)])
    plsc.subcore_barrier()

    pltpu.sync_copy(src_hbm.at[wid, 0], ring.at[0])
    pltpu.sync_copy(ew_hbm.at[wid, 0], ringw.at[0])
    pltpu.async_copy(hp_hbm.at[ring.at[0]], rbuf0, gsem0)
    pltpu.async_copy(src_hbm.at[wid, 1], ring.at[1], isem)
    pltpu.async_copy(ew_hbm.at[wid, 1], ringw.at[1], isem)

    def scale(jp, rbuf):
        @plsc.parallel_loop(0, EB // 16, 1, unroll=2)
        def grp(g):
            nv = ringw[jp, pl.ds(g * 16, 16)]
            for i in range(16):
                sp = _splat(nv, i)
                e = g * 16 + i
                for f in range(8):
                    rbuf[e, pl.ds(f * 16, 16)] = rbuf[e, pl.ds(f * 16, 16)] * sp

    def step(j, rbuf, obuf, osem, ssem_self, ssem_other):
        p = j % 2

        @pl.when(j + 1 < nb)
        def _():
            pltpu.make_async_copy(src_hbm.at[wid, j + 1], ring.at[1 - p],
                                  isem).wait()
            pltpu.make_async_copy(ew_hbm.at[wid, j + 1], ringw.at[1 - p],
                                  isem).wait()

            @pl.when(j >= 1)
            def _():
                pltpu.make_async_copy(obuf, acc_sh.at[dstb.at[j - 1]],
                                      ssem_other).wait()

            pltpu.async_copy(hp_hbm.at[ring.at[1 - p]], obuf, osem)

        scale(p, rbuf)

        @pl.when(j + 2 < nb)
        def _():
            pltpu.async_copy(src_hbm.at[wid, j + 2], ring.at[p], isem)
            pltpu.async_copy(ew_hbm.at[wid, j + 2], ringw.at[p], isem)

        pltpu.async_copy(rbuf, acc_sh.at[dstb.at[j]], ssem_self, add=True)

    def pair(jp, carry):
        j0 = jp * 2
        pltpu.make_async_copy(hp_hbm.at[ring.at[0]], rbuf0, gsem0).wait()
        step(j0, rbuf0, rbuf1, gsem1, ssem0, ssem1)
        pltpu.make_async_copy(hp_hbm.at[ring.at[1]], rbuf1, gsem1).wait()
        step(j0 + 1, rbuf1, rbuf0, gsem0, ssem1, ssem0)
        return carry

    lax.fori_loop(0, nb // 2, pair, 0)

    pltpu.make_async_copy(rbuf0, acc_sh.at[dstb.at[nb - 2]], ssem0).wait()
    pltpu.make_async_copy(rbuf1, acc_sh.at[dstb.at[nb - 1]], ssem1).wait()

    plsc.subcore_barrier()
    pltpu.sync_copy(acc_sh.at[pl.ds(s * RPT, RPT)],
                    out_hbm.at[c, pl.ds(s * RPT, RPT)])


@functools.cache
def _scatter_kernel():
    return pl.kernel(
        _scatter_body,
        out_type=jax.ShapeDtypeStruct((NCORES, NP, D), jnp.float32),
        mesh=_mesh(),
        scratch_types=[
            pltpu.VMEM_SHARED((NP, D), jnp.float32),
            pltpu.VMEM((NBMAX, EB), jnp.int32),
            pltpu.VMEM((2, EB), jnp.int32),
            pltpu.VMEM((2, EB), jnp.float32),
            pltpu.VMEM((EB, D), jnp.float32),
            pltpu.VMEM((EB, D), jnp.float32),
            pltpu.SemaphoreType.DMA,
            pltpu.SemaphoreType.DMA,
            pltpu.SemaphoreType.DMA,
            pltpu.SemaphoreType.DMA,
            pltpu.SemaphoreType.DMA,
        ],
    )


_BLK = 1000
_NBLK = N // _BLK


def _dot(a, b):
    return jnp.dot(a, b, preferred_element_type=jnp.float32,
                   precision=lax.Precision.HIGHEST)


def _prep_body(d0_ref, d1_ref, x_ref, w_ref, disc_ref, o_ref):
    disc = lax.rsqrt(d0_ref[...] + d1_ref[...])
    disc_ref[...] = disc
    o_ref[...] = disc * _dot(x_ref[...], w_ref[...])


def _tc_prep(d0, d1, x, w):
    return pl.pallas_call(
        _prep_body,
        grid=(_NBLK,),
        in_specs=[
            pl.BlockSpec((_BLK, 1), lambda i: (i, 0)),
            pl.BlockSpec((_BLK, 1), lambda i: (i, 0)),
            pl.BlockSpec((_BLK, D), lambda i: (i, 0)),
            pl.BlockSpec((D, D), lambda i: (0, 0)),
        ],
        out_specs=[
            pl.BlockSpec((_BLK, 1), lambda i: (i, 0)),
            pl.BlockSpec((_BLK, D), lambda i: (i, 0)),
        ],
        out_shape=[
            jax.ShapeDtypeStruct((N, 1), jnp.float32),
            jax.ShapeDtypeStruct((N, D), jnp.float32),
        ],
    )(d0, d1, x, w)


def _bn_from(sacc, t, g_ref, be_ref):
    mu = sacc[0:1, :] * (1.0 / N)
    var = sacc[1:2, :] * (1.0 / N) - mu * mu
    return (t - mu) * lax.rsqrt(var + 1e-5) * g_ref[...] + be_ref[...]


def _layer_body(a0_ref, a1_ref, disc_ref, b_ref, g_ref, be_ref, w_ref,
                o_ref, tbuf, sacc, *, relu):
    i = pl.program_id(0)

    @pl.when(i == 0)
    def _():
        sacc[...] = jnp.zeros_like(sacc)

    @pl.when(i < _NBLK)
    def _():
        t = disc_ref[...] * (a0_ref[...] + a1_ref[...]) + b_ref[...]
        if relu:
            t = jnp.maximum(t, 0.0)
        k = i * _BLK
        tbuf[pl.ds(k, _BLK), :] = t
        sacc[0:1, :] += jnp.sum(t, axis=0, keepdims=True)
        sacc[1:2, :] += jnp.sum(t * t, axis=0, keepdims=True)

    @pl.when(i >= _NBLK)
    def _():
        k = (i - _NBLK) * _BLK
        xn = _bn_from(sacc, tbuf[pl.ds(k, _BLK), :], g_ref, be_ref)
        o_ref[...] = disc_ref[...] * _dot(xn, w_ref[...])


def _tc_layer(a0, a1, disc, bias, g, be, w, relu):
    return pl.pallas_call(
        functools.partial(_layer_body, relu=relu),
        grid=(2 * _NBLK,),
        in_specs=[
            pl.BlockSpec((_BLK, D), lambda i: (i % _NBLK, 0)),
            pl.BlockSpec((_BLK, D), lambda i: (i % _NBLK, 0)),
            pl.BlockSpec((_BLK, 1), lambda i: (i % _NBLK, 0)),
            pl.BlockSpec((1, D), lambda i: (0, 0)),
            pl.BlockSpec((1, D), lambda i: (0, 0)),
            pl.BlockSpec((1, D), lambda i: (0, 0)),
            pl.BlockSpec((D, D), lambda i: (0, 0)),
        ],
        out_specs=pl.BlockSpec((_BLK, D), lambda i: (i % _NBLK, 0)),
        out_shape=jax.ShapeDtypeStruct((N, D), jnp.float32),
        scratch_shapes=[
            pltpu.VMEM((N, D), jnp.float32),
            pltpu.VMEM((2, D), jnp.float32),
        ],
    )(a0, a1, disc, bias, g, be, w)


def _tail_body(a0_ref, a1_ref, disc_ref, b_ref, g_ref, be_ref, bat_ref,
               seq_ref, fw_ref, fb_ref, lw_ref, lb_ref, o_ref,
               tbuf, sacc, pool_s, cnt_s):
    i = pl.program_id(0)

    @pl.when(i == 0)
    def _():
        sacc[...] = jnp.zeros_like(sacc)
        pool_s[...] = jnp.zeros_like(pool_s)
        for cc in range(B):
            cnt_s[0, cc] = 0.0

    @pl.when(i < _NBLK)
    def _():
        t = disc_ref[...] * (a0_ref[...] + a1_ref[...]) + b_ref[...]
        k = i * _BLK
        tbuf[pl.ds(k, _BLK), :] = t
        sacc[0:1, :] += jnp.sum(t, axis=0, keepdims=True)
        sacc[1:2, :] += jnp.sum(t * t, axis=0, keepdims=True)

    @pl.when(i >= _NBLK)
    def _():
        k = (i - _NBLK) * _BLK
        xn = _bn_from(sacc, tbuf[pl.ds(k, _BLK), :], g_ref, be_ref)
        bat = bat_ref[0]
        for cc in range(B):
            m = (bat == cc).astype(jnp.float32)
            pool_s[cc:cc + 1, :] += _dot(m, xn)
            cnt_s[0, cc] += jnp.sum(m)

    @pl.when(i == 2 * _NBLK - 1)
    def _():
        for cc in range(B):
            inv = 1.0 / jnp.maximum(cnt_s[0, cc], 1.0)
            pool_s[cc:cc + 1, :] *= inv
        z = pool_s[...] + _dot(seq_ref[...], fw_ref[...]) + fb_ref[...]
        o_ref[...] = jax.nn.sigmoid(_dot(z, lw_ref[...]) + lb_ref[...])


def _tc_tail(a0, a1, disc, bias, g, be, bat3d, seq, fw, fb, lw, lb):
    return pl.pallas_call(
        _tail_body,
        grid=(2 * _NBLK,),
        in_specs=[
            pl.BlockSpec((_BLK, D), lambda i: (i % _NBLK, 0)),
            pl.BlockSpec((_BLK, D), lambda i: (i % _NBLK, 0)),
            pl.BlockSpec((_BLK, 1), lambda i: (i % _NBLK, 0)),
            pl.BlockSpec((1, D), lambda i: (0, 0)),
            pl.BlockSpec((1, D), lambda i: (0, 0)),
            pl.BlockSpec((1, D), lambda i: (0, 0)),
            pl.BlockSpec((1, 1, _BLK), lambda i: (i % _NBLK, 0, 0)),
            pl.BlockSpec((B, SEQ_D), lambda i: (0, 0)),
            pl.BlockSpec((SEQ_D, D), lambda i: (0, 0)),
            pl.BlockSpec((1, D), lambda i: (0, 0)),
            pl.BlockSpec((D, D), lambda i: (0, 0)),
            pl.BlockSpec((1, D), lambda i: (0, 0)),
        ],
        out_specs=pl.BlockSpec((B, D), lambda i: (0, 0)),
        out_shape=jax.ShapeDtypeStruct((B, D), jnp.float32),
        scratch_shapes=[
            pltpu.VMEM((N, D), jnp.float32),
            pltpu.VMEM((2, D), jnp.float32),
            pltpu.VMEM((B, D), jnp.float32),
            pltpu.SMEM((1, B), jnp.float32),
        ],
    )(a0, a1, disc, bias, g, be, bat3d, seq, fw, fb, lw, lb)


def kernel(embedding_features_per_residue, edge_index, edge_attr, batch,
           embedding_features_per_sequence, W1, b1, W2, b2, W3, b3,
           g1, be1, g2, be2, g3, be3, fc1_W, fc1_b, lin_W, lin_b):
    x = embedding_features_per_residue
    src = edge_index[0]
    dst = edge_index[1]
    ew = edge_attr[:, 0]

    loop = jnp.arange(N, dtype=jnp.int32)
    padi = jnp.zeros((EPAD - E - N,), jnp.int32)
    srcF = jnp.concatenate([src, loop, padi])
    dstF = jnp.concatenate([dst, loop, padi])
    ewF = jnp.concatenate([ew, jnp.ones((N,), jnp.float32),
                           jnp.zeros((EPAD - E - N,), jnp.float32)])

    def _tileize(flat):
        arr = jnp.zeros((NTILES, NBMAX, EB), flat.dtype)
        off = 0
        for t in range(NTILES):
            nb = NB0 if t < NSUB else NB1
            arr = arr.at[t, :nb].set(flat[off:off + nb * EB].reshape(nb, EB))
            off += nb * EB
        return arr

    srcA = _tileize(srcF)
    dstA = _tileize(dstF)
    ewA = _tileize(ewF)

    dega = _deg_kernel()(dstA, ewA)
    d0 = dega[0, :N].reshape(N, 1)
    d1 = dega[1, :N].reshape(N, 1)

    b1r, b2r, b3r = (v.reshape(1, D) for v in (b1, b2, b3))
    g1r, g2r, g3r = (v.reshape(1, D) for v in (g1, g2, g3))
    be1r, be2r, be3r = (v.reshape(1, D) for v in (be1, be2, be3))

    disc, h = _tc_prep(d0, d1, x, W1)

    acc = _scatter_kernel()(h, srcA, ewA, dstA)
    h = _tc_layer(acc[0, :N], acc[1, :N], disc, b1r, g1r, be1r, W2, relu=True)

    acc = _scatter_kernel()(h, srcA, ewA, dstA)
    h = _tc_layer(acc[0, :N], acc[1, :N], disc, b2r, g2r, be2r, W3, relu=True)

    acc = _scatter_kernel()(h, srcA, ewA, dstA)

    bat3d = batch.reshape(_NBLK, 1, _BLK)
    lwp = jnp.zeros((D, D), jnp.float32).at[:, :NCLS].set(lin_W)
    lbp = jnp.zeros((1, D), jnp.float32).at[0, :NCLS].set(lin_b)
    out = _tc_tail(acc[0, :N], acc[1, :N], disc, b3r, g3r, be3r, bat3d,
                   embedding_features_per_sequence, fc1_W,
                   fc1_b.reshape(1, D), lwp, lbp)
    return out[:, :NCLS]
